# Initial kernel scaffold; baseline (speedup 1.0000x reference)
import jax
import jax.numpy as jnp
from jax.experimental import pallas as pl


def kernel(conv1_w, conv1_b, conv1_s, conv1_t, conv2_w, conv2_b, conv2_s, conv2_t, conv3_w, conv3_b, conv4_w, conv4_b, conv5_w, conv5_b, fc1_w, fc1_b, fc2_w, fc2_b, fc3_w, fc3_b, x):
    raise NotImplementedError("write your pallas kernel here")


# speedup vs baseline: 1.7631x; 1.7631x over previous
import functools

import jax
import jax.numpy as jnp
from jax.experimental import pallas as pl
from jax.experimental.pallas import tpu as pltpu

_NUM_CLASSES = 4
_VMEM_LIMIT = 48 * 1024 * 1024


def _ru(a, b):
    return ((a + b - 1) // b) * b


def _conv_body(x_ref, w_ref, b_ref, o_ref, *, groups):
    for g in range(groups):
        y = jnp.dot(x_ref[g], w_ref[g], preferred_element_type=jnp.float32)
        o_ref[g] = jnp.maximum(y + b_ref[g], 0.0).astype(o_ref.dtype)


def _conv_bn_body(x_ref, w_ref, b_ref, s_ref, t_ref, o_ref, *, groups):
    for g in range(groups):
        y = jnp.dot(x_ref[g], w_ref[g], preferred_element_type=jnp.float32)
        y = jnp.maximum(y + b_ref[g], 0.0)
        o_ref[g] = (y * s_ref[g] + t_ref[g]).astype(o_ref.dtype)


def _fc_body(x_ref, w_ref, b_ref, o_ref, *, relu):
    y = jnp.dot(x_ref[...], w_ref[...], preferred_element_type=jnp.float32)
    y = y + b_ref[...]
    if relu:
        y = jnp.maximum(y, 0.0)
    o_ref[...] = y.astype(o_ref.dtype)


def _pool_body(x_ref, o_ref):
    o_ref[...] = jnp.max(x_ref[...], axis=0)


def _conv_gemm(patches, w, b, s=None, t=None, *, tm=2048):
    G, M, K = patches.shape
    N = w.shape[2]
    Mp = _ru(M, tm)
    xp = jnp.pad(patches, ((0, 0), (0, Mp - M), (0, 0)))

    vec = pl.BlockSpec((G, 1, N), lambda i: (0, 0, 0))
    in_specs = [
        pl.BlockSpec((G, tm, K), lambda i: (0, i, 0)),
        pl.BlockSpec((G, K, N), lambda i: (0, 0, 0)),
        vec,
    ]
    args = [xp, w, b]
    body = functools.partial(_conv_body, groups=G)
    if s is not None:
        in_specs += [vec, vec]
        args += [s, t]
        body = functools.partial(_conv_bn_body, groups=G)

    out = pl.pallas_call(
        body,
        out_shape=jax.ShapeDtypeStruct((G, Mp, N), jnp.bfloat16),
        grid=(Mp // tm,),
        in_specs=in_specs,
        out_specs=pl.BlockSpec((G, tm, N), lambda i: (0, i, 0)),
        compiler_params=pltpu.CompilerParams(
            dimension_semantics=("parallel",),
            vmem_limit_bytes=_VMEM_LIMIT),
    )(*args)
    return out[:, :M, :]


def _fc(x, w, b, *, relu, out_dtype):
    M, K = x.shape
    Kp, Np = w.shape
    tn = 512 if Np % 512 == 0 else Np
    Mp = _ru(M, 16)
    xp = jnp.pad(x, ((0, Mp - M), (0, Kp - K)))

    out = pl.pallas_call(
        functools.partial(_fc_body, relu=relu),
        out_shape=jax.ShapeDtypeStruct((Mp, Np), out_dtype),
        grid=(Np // tn,),
        in_specs=[
            pl.BlockSpec((Mp, Kp), lambda j: (0, 0)),
            pl.BlockSpec((Kp, tn), lambda j: (0, j)),
            pl.BlockSpec((1, tn), lambda j: (0, j)),
        ],
        out_specs=pl.BlockSpec((Mp, tn), lambda j: (0, j)),
        compiler_params=pltpu.CompilerParams(
            dimension_semantics=("parallel",),
            vmem_limit_bytes=_VMEM_LIMIT),
    )(xp, w, b)
    return out[:M, :]


def _maxpool_3x3_s2(x):
    B, H, W, C = x.shape
    OH = (H - 3) // 2 + 1
    OW = (W - 3) // 2 + 1
    slabs = [x[:, dy:dy + 2 * (OH - 1) + 1:2,
               dx:dx + 2 * (OW - 1) + 1:2, :].reshape(-1)
             for dy in range(3) for dx in range(3)]
    stacked = jnp.stack(slabs, axis=0)
    L = B * OH * OW * C
    Q = -(-L // 128)
    tq = min(512, _ru(Q, 16))
    Qp = _ru(Q, tq)
    stacked = jnp.pad(stacked, ((0, 0), (0, Qp * 128 - L)))
    stacked = stacked.reshape(9, Qp, 128)

    out = pl.pallas_call(
        _pool_body,
        out_shape=jax.ShapeDtypeStruct((Qp, 128), x.dtype),
        grid=(Qp // tq,),
        in_specs=[pl.BlockSpec((9, tq, 128), lambda i: (0, i, 0))],
        out_specs=pl.BlockSpec((tq, 128), lambda i: (i, 0)),
        compiler_params=pltpu.CompilerParams(
            dimension_semantics=("parallel",),
            vmem_limit_bytes=_VMEM_LIMIT),
    )(stacked)
    return out.reshape(-1)[:L].reshape(B, OH, OW, C)


def _im2col(x, kh, kw, stride, pad):
    if pad:
        x = jnp.pad(x, ((0, 0), (pad, pad), (pad, pad), (0, 0)))
    B, H, W, C = x.shape
    OH = (H - kh) // stride + 1
    OW = (W - kw) // stride + 1
    cols = []
    for dy in range(kh):
        for dx in range(kw):
            cols.append(x[:, dy:dy + stride * (OH - 1) + 1:stride,
                          dx:dx + stride * (OW - 1) + 1:stride, :])
    patches = jnp.stack(cols, axis=3)
    return patches.reshape(B * OH * OW, kh * kw * C), (OH, OW)


def _block_diag_w(w, groups, kg, ng, n_pad):
    g = w.shape[0]
    wt = w[:, :9 * kg, :ng].reshape(g, 9, kg, ng)
    wt = jnp.transpose(wt, (1, 0, 2, 3))
    eye = jnp.eye(g, dtype=w.dtype)
    blk = wt[:, :, :, None, :] * eye[None, :, None, :, None]
    blk = blk.reshape(9 * g * kg, g * ng)
    if n_pad:
        blk = jnp.pad(blk, ((0, 0), (0, n_pad)))
    return blk


def _cat_vec(v, ng, n_pad, fill=0.0):
    g = v.shape[0]
    flat = v[:, 0, :ng].reshape(1, g * ng)
    if n_pad:
        flat = jnp.pad(flat, ((0, 0), (0, n_pad)), constant_values=fill)
    return flat


def kernel(conv1_w, conv1_b, conv1_s, conv1_t,
           conv2_w, conv2_b, conv2_s, conv2_t,
           conv3_w, conv3_b, conv4_w, conv4_b, conv5_w, conv5_b,
           fc1_w, fc1_b, fc2_w, fc2_b, fc3_w, fc3_b, x):
    bf = jnp.bfloat16
    xh = jnp.transpose(x, (0, 2, 3, 1)).astype(bf)
    n = xh.shape[0]

    w1 = jnp.pad(conv1_w[0, :363, :96], ((0, 0), (0, 160))).astype(bf)
    b1 = jnp.pad(conv1_b[0, :, :96], ((0, 0), (0, 160)))
    s1 = jnp.pad(conv1_s[0, :, :96], ((0, 0), (0, 160)), constant_values=1.0)
    t1 = jnp.pad(conv1_t[0, :, :96], ((0, 0), (0, 160)))
    p1, (oh1, ow1) = _im2col(xh, 11, 11, 4, 0)
    c1 = _conv_gemm(p1[None], w1[None], b1[None], s1[None], t1[None])
    c1 = c1[0, :, :96].reshape(n, oh1, ow1, 96)
    c1 = _maxpool_3x3_s2(c1)

    w2 = _block_diag_w(conv2_w.astype(bf), 4, 24, 64, 0)
    b2 = _cat_vec(conv2_b, 64, 0)
    s2 = _cat_vec(conv2_s, 64, 0, fill=1.0)
    t2 = _cat_vec(conv2_t, 64, 0)
    p2, (oh, ow) = _im2col(c1, 3, 3, 1, 1)
    c2 = _conv_gemm(p2[None], w2[None], b2[None], s2[None], t2[None])
    c2 = c2[0].reshape(n, oh, ow, 256)
    c2 = _maxpool_3x3_s2(c2)

    w3 = _block_diag_w(conv3_w.astype(bf), 2, 128, 192, 0)
    b3 = _cat_vec(conv3_b, 192, 0)
    p3, (oh, ow) = _im2col(c2, 3, 3, 1, 1)
    c3 = _conv_gemm(p3[None], w3[None], b3[None])
    c3 = c3[0].reshape(n, oh, ow, 384)

    w4 = conv4_w.astype(bf)
    w4 = jnp.stack([_block_diag_w(w4[0:2], 2, 96, 96, 64),
                    _block_diag_w(w4[2:4], 2, 96, 96, 64)])
    b4 = jnp.stack([_cat_vec(conv4_b[0:2], 96, 64),
                    _cat_vec(conv4_b[2:4], 96, 64)])
    pa, _ = _im2col(c3[..., :192], 3, 3, 1, 1)
    pb, _ = _im2col(c3[..., 192:], 3, 3, 1, 1)
    p4 = jnp.stack([pa, pb])
    c4 = _conv_gemm(p4, w4, b4)
    M = n * oh * ow
    c4 = jnp.concatenate([c4[0, :M, :192], c4[1, :M, :192]], axis=1)
    c4 = c4.reshape(n, oh, ow, 384)

    w5 = conv5_w.astype(bf)
    w5 = jnp.stack([_block_diag_w(w5[0:2], 2, 96, 64, 128),
                    _block_diag_w(w5[2:4], 2, 96, 64, 128)])
    b5 = jnp.stack([_cat_vec(conv5_b[0:2], 64, 128),
                    _cat_vec(conv5_b[2:4], 64, 128)])
    pa, _ = _im2col(c4[..., :192], 3, 3, 1, 1)
    pb, _ = _im2col(c4[..., 192:], 3, 3, 1, 1)
    p5 = jnp.stack([pa, pb])
    c5 = _conv_gemm(p5, w5, b5)
    c5 = jnp.concatenate([c5[0, :M, :128], c5[1, :M, :128]], axis=1)
    c5 = c5.reshape(n, oh, ow, 256)
    c5 = _maxpool_3x3_s2(c5)

    flat = jnp.transpose(c5, (0, 3, 1, 2)).reshape(n, -1)

    h1 = _fc(flat, fc1_w, fc1_b, relu=True, out_dtype=bf)
    h2 = _fc(h1, fc2_w, fc2_b, relu=True, out_dtype=bf)
    out = _fc(h2, fc3_w, fc3_b, relu=False, out_dtype=jnp.float32)
    return out[:, :_NUM_CLASSES]


# speedup vs baseline: 2.7832x; 1.5786x over previous
import functools

import jax
import jax.numpy as jnp
from jax.experimental import pallas as pl
from jax.experimental.pallas import tpu as pltpu

_NUM_CLASSES = 4
_VMEM_LIMIT = 48 * 1024 * 1024


def _ru(a, b):
    return ((a + b - 1) // b) * b


def _conv_body(x_ref, w_ref, b_ref, o_ref, *, groups):
    for g in range(groups):
        y = jnp.dot(x_ref[g], w_ref[g], preferred_element_type=jnp.float32)
        o_ref[g] = jnp.maximum(y + b_ref[g], 0.0).astype(o_ref.dtype)


def _conv_bn_body(x_ref, w_ref, b_ref, s_ref, t_ref, o_ref, *, groups):
    for g in range(groups):
        y = jnp.dot(x_ref[g], w_ref[g], preferred_element_type=jnp.float32)
        y = jnp.maximum(y + b_ref[g], 0.0)
        o_ref[g] = (y * s_ref[g] + t_ref[g]).astype(o_ref.dtype)


def _fc_body(x_ref, w_ref, b_ref, o_ref, *, relu):
    y = jnp.dot(x_ref[...], w_ref[...], preferred_element_type=jnp.float32)
    y = y + b_ref[...]
    if relu:
        y = jnp.maximum(y, 0.0)
    o_ref[...] = y.astype(o_ref.dtype)


def _pool_body(ee_ref, eo_ref, oe_ref, oo_ref, o_ref):
    oh = o_ref.shape[1]
    ow = o_ref.shape[2]
    ee = ee_ref[...]
    eo = eo_ref[...]
    oe = oe_ref[...]
    m = jnp.maximum
    top = m(m(ee[:, :oh, :ow], eo[:, :oh, :ow]), ee[:, :oh, 1:ow + 1])
    mid = m(m(oe[:, :oh, :ow], oo_ref[...]), oe[:, :oh, 1:ow + 1])
    bot = m(m(ee[:, 1:oh + 1, :ow], eo[:, 1:oh + 1, :ow]),
            ee[:, 1:oh + 1, 1:ow + 1])
    o_ref[...] = m(m(top, mid), bot)


def _conv_gemm(patches, w, b, s=None, t=None, *, tm=2048):
    G, M, K = patches.shape
    N = w.shape[2]
    Mp = _ru(M, tm)
    xp = jnp.pad(patches, ((0, 0), (0, Mp - M), (0, 0)))

    vec = pl.BlockSpec((G, 1, N), lambda i: (0, 0, 0))
    in_specs = [
        pl.BlockSpec((G, tm, K), lambda i: (0, i, 0)),
        pl.BlockSpec((G, K, N), lambda i: (0, 0, 0)),
        vec,
    ]
    args = [xp, w, b]
    body = functools.partial(_conv_body, groups=G)
    if s is not None:
        in_specs += [vec, vec]
        args += [s, t]
        body = functools.partial(_conv_bn_body, groups=G)

    out = pl.pallas_call(
        body,
        out_shape=jax.ShapeDtypeStruct((G, Mp, N), jnp.bfloat16),
        grid=(Mp // tm,),
        in_specs=in_specs,
        out_specs=pl.BlockSpec((G, tm, N), lambda i: (0, i, 0)),
        compiler_params=pltpu.CompilerParams(
            dimension_semantics=("parallel",),
            vmem_limit_bytes=_VMEM_LIMIT),
    )(*args)
    return out[:, :M, :]


def _fc(x, w, b, *, relu, out_dtype):
    M, K = x.shape
    Kp, Np = w.shape
    tn = 512 if Np % 512 == 0 else Np
    Mp = _ru(M, 16)
    xp = jnp.pad(x, ((0, Mp - M), (0, Kp - K)))

    out = pl.pallas_call(
        functools.partial(_fc_body, relu=relu),
        out_shape=jax.ShapeDtypeStruct((Mp, Np), out_dtype),
        grid=(Np // tn,),
        in_specs=[
            pl.BlockSpec((Mp, Kp), lambda j: (0, 0)),
            pl.BlockSpec((Kp, tn), lambda j: (0, j)),
            pl.BlockSpec((1, tn), lambda j: (0, j)),
        ],
        out_specs=pl.BlockSpec((Mp, tn), lambda j: (0, j)),
        compiler_params=pltpu.CompilerParams(
            dimension_semantics=("parallel",),
            vmem_limit_bytes=_VMEM_LIMIT),
    )(xp, w, b)
    return out[:M, :]


def _maxpool_3x3_s2(x, bt=16):
    B, H, W, C = x.shape
    if B % bt:
        bt = B
    OH = (H - 3) // 2 + 1
    OW = (W - 3) // 2 + 1
    ee = x[:, 0::2, 0::2]
    eo = x[:, 0::2, 1::2]
    oe = x[:, 1::2, 0::2]
    oo = x[:, 1::2, 1::2]

    def spec(a):
        return pl.BlockSpec((bt,) + a.shape[1:], lambda i: (i, 0, 0, 0))

    return pl.pallas_call(
        _pool_body,
        out_shape=jax.ShapeDtypeStruct((B, OH, OW, C), x.dtype),
        grid=(B // bt,),
        in_specs=[spec(ee), spec(eo), spec(oe), spec(oo)],
        out_specs=pl.BlockSpec((bt, OH, OW, C), lambda i: (i, 0, 0, 0)),
        compiler_params=pltpu.CompilerParams(
            dimension_semantics=("parallel",),
            vmem_limit_bytes=_VMEM_LIMIT),
    )(ee, eo, oe, oo)


def _im2col(x, kh, kw, stride, pad):
    if pad:
        x = jnp.pad(x, ((0, 0), (pad, pad), (pad, pad), (0, 0)))
    B, H, W, C = x.shape
    OH = (H - kh) // stride + 1
    OW = (W - kw) // stride + 1
    cols = []
    for dy in range(kh):
        for dx in range(kw):
            cols.append(x[:, dy:dy + stride * (OH - 1) + 1:stride,
                          dx:dx + stride * (OW - 1) + 1:stride, :])
    patches = jnp.stack(cols, axis=3)
    return patches.reshape(B * OH * OW, kh * kw * C), (OH, OW)


def _block_diag_w(w, groups, kg, ng, n_pad):
    g = w.shape[0]
    wt = w[:, :9 * kg, :ng].reshape(g, 9, kg, ng)
    wt = jnp.transpose(wt, (1, 0, 2, 3))
    eye = jnp.eye(g, dtype=w.dtype)
    blk = wt[:, :, :, None, :] * eye[None, :, None, :, None]
    blk = blk.reshape(9 * g * kg, g * ng)
    if n_pad:
        blk = jnp.pad(blk, ((0, 0), (0, n_pad)))
    return blk


def _cat_vec(v, ng, n_pad, fill=0.0):
    g = v.shape[0]
    flat = v[:, 0, :ng].reshape(1, g * ng)
    if n_pad:
        flat = jnp.pad(flat, ((0, 0), (0, n_pad)), constant_values=fill)
    return flat


def kernel(conv1_w, conv1_b, conv1_s, conv1_t,
           conv2_w, conv2_b, conv2_s, conv2_t,
           conv3_w, conv3_b, conv4_w, conv4_b, conv5_w, conv5_b,
           fc1_w, fc1_b, fc2_w, fc2_b, fc3_w, fc3_b, x):
    bf = jnp.bfloat16
    xh = jnp.transpose(x, (0, 2, 3, 1)).astype(bf)
    n = xh.shape[0]

    w1 = jnp.pad(conv1_w[0, :363, :96], ((0, 0), (0, 160))).astype(bf)
    b1 = jnp.pad(conv1_b[0, :, :96], ((0, 0), (0, 160)))
    s1 = jnp.pad(conv1_s[0, :, :96], ((0, 0), (0, 160)), constant_values=1.0)
    t1 = jnp.pad(conv1_t[0, :, :96], ((0, 0), (0, 160)))
    p1, (oh1, ow1) = _im2col(xh, 11, 11, 4, 0)
    c1 = _conv_gemm(p1[None], w1[None], b1[None], s1[None], t1[None])
    c1 = c1[0, :, :96].reshape(n, oh1, ow1, 96)
    c1 = _maxpool_3x3_s2(c1)

    w2 = _block_diag_w(conv2_w.astype(bf), 4, 24, 64, 0)
    b2 = _cat_vec(conv2_b, 64, 0)
    s2 = _cat_vec(conv2_s, 64, 0, fill=1.0)
    t2 = _cat_vec(conv2_t, 64, 0)
    p2, (oh, ow) = _im2col(c1, 3, 3, 1, 1)
    c2 = _conv_gemm(p2[None], w2[None], b2[None], s2[None], t2[None])
    c2 = c2[0].reshape(n, oh, ow, 256)
    c2 = _maxpool_3x3_s2(c2)

    w3 = _block_diag_w(conv3_w.astype(bf), 2, 128, 192, 0)
    b3 = _cat_vec(conv3_b, 192, 0)
    p3, (oh, ow) = _im2col(c2, 3, 3, 1, 1)
    c3 = _conv_gemm(p3[None], w3[None], b3[None])
    c3 = c3[0].reshape(n, oh, ow, 384)

    w4 = conv4_w.astype(bf)
    w4 = jnp.stack([_block_diag_w(w4[0:2], 2, 96, 96, 64),
                    _block_diag_w(w4[2:4], 2, 96, 96, 64)])
    b4 = jnp.stack([_cat_vec(conv4_b[0:2], 96, 64),
                    _cat_vec(conv4_b[2:4], 96, 64)])
    pa, _ = _im2col(c3[..., :192], 3, 3, 1, 1)
    pb, _ = _im2col(c3[..., 192:], 3, 3, 1, 1)
    p4 = jnp.stack([pa, pb])
    c4 = _conv_gemm(p4, w4, b4)
    M = n * oh * ow
    c4 = jnp.concatenate([c4[0, :M, :192], c4[1, :M, :192]], axis=1)
    c4 = c4.reshape(n, oh, ow, 384)

    w5 = conv5_w.astype(bf)
    w5 = jnp.stack([_block_diag_w(w5[0:2], 2, 96, 64, 128),
                    _block_diag_w(w5[2:4], 2, 96, 64, 128)])
    b5 = jnp.stack([_cat_vec(conv5_b[0:2], 64, 128),
                    _cat_vec(conv5_b[2:4], 64, 128)])
    pa, _ = _im2col(c4[..., :192], 3, 3, 1, 1)
    pb, _ = _im2col(c4[..., 192:], 3, 3, 1, 1)
    p5 = jnp.stack([pa, pb])
    c5 = _conv_gemm(p5, w5, b5)
    c5 = jnp.concatenate([c5[0, :M, :128], c5[1, :M, :128]], axis=1)
    c5 = c5.reshape(n, oh, ow, 256)
    c5 = _maxpool_3x3_s2(c5)

    flat = jnp.transpose(c5, (0, 3, 1, 2)).reshape(n, -1)

    h1 = _fc(flat, fc1_w, fc1_b, relu=True, out_dtype=bf)
    h2 = _fc(h1, fc2_w, fc2_b, relu=True, out_dtype=bf)
    out = _fc(h2, fc3_w, fc3_b, relu=False, out_dtype=jnp.float32)
    return out[:, :_NUM_CLASSES]


# speedup vs baseline: 4.1768x; 1.5007x over previous
import functools

import jax
import jax.numpy as jnp
from jax.experimental import pallas as pl
from jax.experimental.pallas import tpu as pltpu

_NUM_CLASSES = 4
_VMEM_LIMIT = 48 * 1024 * 1024


def _ru(a, b):
    return ((a + b - 1) // b) * b


def _conv_body(x_ref, w_ref, b_ref, o_ref, *, groups):
    for g in range(groups):
        y = jnp.dot(x_ref[g], w_ref[g], preferred_element_type=jnp.float32)
        o_ref[g] = jnp.maximum(y + b_ref[g], 0.0).astype(o_ref.dtype)


def _conv_bn_body(x_ref, w_ref, b_ref, s_ref, t_ref, o_ref, *, groups):
    for g in range(groups):
        y = jnp.dot(x_ref[g], w_ref[g], preferred_element_type=jnp.float32)
        y = jnp.maximum(y + b_ref[g], 0.0)
        o_ref[g] = (y * s_ref[g] + t_ref[g]).astype(o_ref.dtype)


def _fc_body(x_ref, w_ref, b_ref, o_ref, *, relu):
    y = jnp.dot(x_ref[...], w_ref[...], preferred_element_type=jnp.float32)
    y = y + b_ref[...]
    if relu:
        y = jnp.maximum(y, 0.0)
    o_ref[...] = y.astype(o_ref.dtype)


def _pool_body(ee_ref, eo_ref, oe_ref, oo_ref, o_ref):
    oh = o_ref.shape[1]
    ow = o_ref.shape[2]
    ee = ee_ref[...]
    eo = eo_ref[...]
    oe = oe_ref[...]
    m = jnp.maximum
    top = m(m(ee[:, :oh, :ow], eo[:, :oh, :ow]), ee[:, :oh, 1:ow + 1])
    mid = m(m(oe[:, :oh, :ow], oo_ref[...]), oe[:, :oh, 1:ow + 1])
    bot = m(m(ee[:, 1:oh + 1, :ow], eo[:, 1:oh + 1, :ow]),
            ee[:, 1:oh + 1, 1:ow + 1])
    o_ref[...] = m(m(top, mid), bot)


def _conv_gemm(patches, w, b, s=None, t=None, *, tm=2048):
    G, M, K = patches.shape
    N = w.shape[2]
    Mp = _ru(M, tm)
    xp = jnp.pad(patches, ((0, 0), (0, Mp - M), (0, 0)))

    vec = pl.BlockSpec((G, 1, N), lambda i: (0, 0, 0))
    in_specs = [
        pl.BlockSpec((G, tm, K), lambda i: (0, i, 0)),
        pl.BlockSpec((G, K, N), lambda i: (0, 0, 0)),
        vec,
    ]
    args = [xp, w, b]
    body = functools.partial(_conv_body, groups=G)
    if s is not None:
        in_specs += [vec, vec]
        args += [s, t]
        body = functools.partial(_conv_bn_body, groups=G)

    out = pl.pallas_call(
        body,
        out_shape=jax.ShapeDtypeStruct((G, Mp, N), jnp.bfloat16),
        grid=(Mp // tm,),
        in_specs=in_specs,
        out_specs=pl.BlockSpec((G, tm, N), lambda i: (0, i, 0)),
        compiler_params=pltpu.CompilerParams(
            dimension_semantics=("parallel",),
            vmem_limit_bytes=_VMEM_LIMIT),
    )(*args)
    return out[:, :M, :]


def _fc(x, w, b, *, relu, out_dtype):
    M, K = x.shape
    Kp, Np = w.shape
    tn = 512 if Np % 512 == 0 else Np
    Mp = _ru(M, 16)
    xp = jnp.pad(x, ((0, Mp - M), (0, Kp - K)))

    out = pl.pallas_call(
        functools.partial(_fc_body, relu=relu),
        out_shape=jax.ShapeDtypeStruct((Mp, Np), out_dtype),
        grid=(Np // tn,),
        in_specs=[
            pl.BlockSpec((Mp, Kp), lambda j: (0, 0)),
            pl.BlockSpec((Kp, tn), lambda j: (0, j)),
            pl.BlockSpec((1, tn), lambda j: (0, j)),
        ],
        out_specs=pl.BlockSpec((Mp, tn), lambda j: (0, j)),
        compiler_params=pltpu.CompilerParams(
            dimension_semantics=("parallel",),
            vmem_limit_bytes=_VMEM_LIMIT),
    )(xp, w, b)
    return out[:M, :]


def _maxpool_3x3_s2(x, bt=16):
    B, H, W, C = x.shape
    if B % bt:
        bt = B
    OH = (H - 3) // 2 + 1
    OW = (W - 3) // 2 + 1
    ee = x[:, 0::2, 0::2]
    eo = x[:, 0::2, 1::2]
    oe = x[:, 1::2, 0::2]
    oo = x[:, 1::2, 1::2]

    def spec(a):
        return pl.BlockSpec((bt,) + a.shape[1:], lambda i: (i, 0, 0, 0))

    return pl.pallas_call(
        _pool_body,
        out_shape=jax.ShapeDtypeStruct((B, OH, OW, C), x.dtype),
        grid=(B // bt,),
        in_specs=[spec(ee), spec(eo), spec(oe), spec(oo)],
        out_specs=pl.BlockSpec((bt, OH, OW, C), lambda i: (i, 0, 0, 0)),
        compiler_params=pltpu.CompilerParams(
            dimension_semantics=("parallel",),
            vmem_limit_bytes=_VMEM_LIMIT),
    )(ee, eo, oe, oo)


def _im2col(x, kh, kw, stride, pad):
    if pad:
        x = jnp.pad(x, ((0, 0), (pad, pad), (pad, pad), (0, 0)))
    B, H, W, C = x.shape
    OH = (H - kh) // stride + 1
    OW = (W - kw) // stride + 1
    cols = []
    for dy in range(kh):
        for dx in range(kw):
            cols.append(x[:, dy:dy + stride * (OH - 1) + 1:stride,
                          dx:dx + stride * (OW - 1) + 1:stride, :])
    patches = jnp.stack(cols, axis=3)
    return patches.reshape(B * OH * OW, kh * kw * C), (OH, OW)


def _block_diag_w(w, groups, kg, ng, n_pad):
    g = w.shape[0]
    wt = w[:, :9 * kg, :ng].reshape(g, 9, kg, ng)
    wt = jnp.transpose(wt, (1, 0, 2, 3))
    eye = jnp.eye(g, dtype=w.dtype)
    blk = wt[:, :, :, None, :] * eye[None, :, None, :, None]
    blk = blk.reshape(9 * g * kg, g * ng)
    if n_pad:
        blk = jnp.pad(blk, ((0, 0), (0, n_pad)))
    return blk


def _cat_vec(v, ng, n_pad, fill=0.0):
    g = v.shape[0]
    flat = v[:, 0, :ng].reshape(1, g * ng)
    if n_pad:
        flat = jnp.pad(flat, ((0, 0), (0, n_pad)), constant_values=fill)
    return flat


def kernel(conv1_w, conv1_b, conv1_s, conv1_t,
           conv2_w, conv2_b, conv2_s, conv2_t,
           conv3_w, conv3_b, conv4_w, conv4_b, conv5_w, conv5_b,
           fc1_w, fc1_b, fc2_w, fc2_b, fc3_w, fc3_b, x):
    bf = jnp.bfloat16
    xh = jnp.transpose(x, (0, 2, 3, 1)).astype(bf)
    n = xh.shape[0]

    w1 = conv1_w[0, :363, :96].reshape(11, 11, 3, 96)
    w1 = jnp.pad(w1, ((0, 1), (0, 1), (0, 0), (0, 0)))
    w1 = w1.reshape(3, 4, 3, 4, 3, 96).transpose(0, 2, 1, 3, 4, 5)
    w1 = jnp.pad(w1.reshape(432, 96), ((0, 0), (0, 160))).astype(bf)
    b1 = jnp.pad(conv1_b[0, :, :96], ((0, 0), (0, 160)))
    s1 = jnp.pad(conv1_s[0, :, :96], ((0, 0), (0, 160)), constant_values=1.0)
    t1 = jnp.pad(conv1_t[0, :, :96], ((0, 0), (0, 160)))
    xb = jnp.pad(xh, ((0, 0), (0, 1), (0, 1), (0, 0)))
    xb = xb.reshape(n, 49, 4, 49, 4, 3).transpose(0, 1, 3, 2, 4, 5)
    xb = xb.reshape(n, 49, 49, 48)
    p1, (oh1, ow1) = _im2col(xb, 3, 3, 1, 0)
    c1 = _conv_gemm(p1[None], w1[None], b1[None], s1[None], t1[None])
    c1 = c1[0, :, :96].reshape(n, oh1, ow1, 96)
    c1 = _maxpool_3x3_s2(c1)

    w2 = _block_diag_w(conv2_w.astype(bf), 4, 24, 64, 0)
    b2 = _cat_vec(conv2_b, 64, 0)
    s2 = _cat_vec(conv2_s, 64, 0, fill=1.0)
    t2 = _cat_vec(conv2_t, 64, 0)
    p2, (oh, ow) = _im2col(c1, 3, 3, 1, 1)
    c2 = _conv_gemm(p2[None], w2[None], b2[None], s2[None], t2[None])
    c2 = c2[0].reshape(n, oh, ow, 256)
    c2 = _maxpool_3x3_s2(c2)

    w3 = _block_diag_w(conv3_w.astype(bf), 2, 128, 192, 0)
    b3 = _cat_vec(conv3_b, 192, 0)
    p3, (oh, ow) = _im2col(c2, 3, 3, 1, 1)
    c3 = _conv_gemm(p3[None], w3[None], b3[None])
    c3 = c3[0].reshape(n, oh, ow, 384)

    w4 = conv4_w.astype(bf)
    w4 = jnp.stack([_block_diag_w(w4[0:2], 2, 96, 96, 64),
                    _block_diag_w(w4[2:4], 2, 96, 96, 64)])
    b4 = jnp.stack([_cat_vec(conv4_b[0:2], 96, 64),
                    _cat_vec(conv4_b[2:4], 96, 64)])
    pa, _ = _im2col(c3[..., :192], 3, 3, 1, 1)
    pb, _ = _im2col(c3[..., 192:], 3, 3, 1, 1)
    p4 = jnp.stack([pa, pb])
    c4 = _conv_gemm(p4, w4, b4)
    M = n * oh * ow
    c4 = jnp.concatenate([c4[0, :M, :192], c4[1, :M, :192]], axis=1)
    c4 = c4.reshape(n, oh, ow, 384)

    w5 = conv5_w.astype(bf)
    w5 = jnp.stack([_block_diag_w(w5[0:2], 2, 96, 64, 128),
                    _block_diag_w(w5[2:4], 2, 96, 64, 128)])
    b5 = jnp.stack([_cat_vec(conv5_b[0:2], 64, 128),
                    _cat_vec(conv5_b[2:4], 64, 128)])
    pa, _ = _im2col(c4[..., :192], 3, 3, 1, 1)
    pb, _ = _im2col(c4[..., 192:], 3, 3, 1, 1)
    p5 = jnp.stack([pa, pb])
    c5 = _conv_gemm(p5, w5, b5)
    c5 = jnp.concatenate([c5[0, :M, :128], c5[1, :M, :128]], axis=1)
    c5 = c5.reshape(n, oh, ow, 256)
    c5 = _maxpool_3x3_s2(c5)

    flat = jnp.transpose(c5, (0, 3, 1, 2)).reshape(n, -1)

    h1 = _fc(flat, fc1_w, fc1_b, relu=True, out_dtype=bf)
    h2 = _fc(h1, fc2_w, fc2_b, relu=True, out_dtype=bf)
    out = _fc(h2, fc3_w, fc3_b, relu=False, out_dtype=jnp.float32)
    return out[:, :_NUM_CLASSES]


# speedup vs baseline: 5.4266x; 1.2992x over previous
import functools

import jax
import jax.numpy as jnp
from jax.experimental import pallas as pl
from jax.experimental.pallas import tpu as pltpu

_NUM_CLASSES = 4
_VMEM_LIMIT = 48 * 1024 * 1024


def _ru(a, b):
    return ((a + b - 1) // b) * b


def _conv_body(x_ref, w_ref, b_ref, o_ref, *, groups):
    for g in range(groups):
        y = jnp.dot(x_ref[g], w_ref[g], preferred_element_type=jnp.float32)
        o_ref[g] = jnp.maximum(y + b_ref[g], 0.0).astype(o_ref.dtype)


def _conv_bn_body(x_ref, w_ref, b_ref, s_ref, t_ref, o_ref, *, groups):
    for g in range(groups):
        y = jnp.dot(x_ref[g], w_ref[g], preferred_element_type=jnp.float32)
        y = jnp.maximum(y + b_ref[g], 0.0)
        o_ref[g] = (y * s_ref[g] + t_ref[g]).astype(o_ref.dtype)


def _flat_conv_body(x_ref, w_ref, b_ref, m_ref, o_ref):
    bt, ff, c = x_ref.shape
    fw = 13
    r = bt * ff
    x2 = x_ref[...].reshape(r, c)
    xp = jnp.pad(x2, ((fw + 1, fw + 1), (0, 0)))
    acc = None
    for dy in (-1, 0, 1):
        s = fw + 1 + dy * fw - 1
        xc = jnp.concatenate(
            [xp[s:s + r], xp[s + 1:s + 1 + r], xp[s + 2:s + 2 + r]], axis=1)
        y = jnp.dot(xc, w_ref[(dy + 1) * 3 * c:(dy + 2) * 3 * c],
                    preferred_element_type=jnp.float32)
        acc = y if acc is None else acc + y
    y = jnp.maximum(acc + b_ref[...], 0.0) * jnp.tile(m_ref[...], (bt, 1))
    o_ref[...] = y.astype(o_ref.dtype).reshape(o_ref.shape)


def _flat_conv(x, w, b, mask, *, bt=16):
    n, ff, c = x.shape
    if n % bt:
        bt = n
    N = w.shape[1]
    return pl.pallas_call(
        _flat_conv_body,
        out_shape=jax.ShapeDtypeStruct((n, ff, N), jnp.bfloat16),
        grid=(n // bt,),
        in_specs=[
            pl.BlockSpec((bt, ff, c), lambda i: (i, 0, 0)),
            pl.BlockSpec(w.shape, lambda i: (0, 0)),
            pl.BlockSpec((1, N), lambda i: (0, 0)),
            pl.BlockSpec((ff, 1), lambda i: (0, 0)),
        ],
        out_specs=pl.BlockSpec((bt, ff, N), lambda i: (i, 0, 0)),
        compiler_params=pltpu.CompilerParams(
            dimension_semantics=("parallel",),
            vmem_limit_bytes=_VMEM_LIMIT),
    )(x, w, b, mask)


def _fc_body(x_ref, w_ref, b_ref, o_ref, *, relu):
    y = jnp.dot(x_ref[...], w_ref[...], preferred_element_type=jnp.float32)
    y = y + b_ref[...]
    if relu:
        y = jnp.maximum(y, 0.0)
    o_ref[...] = y.astype(o_ref.dtype)


def _pool_body(ee_ref, eo_ref, oe_ref, oo_ref, o_ref):
    oh = o_ref.shape[1]
    ow = o_ref.shape[2]
    ee = ee_ref[...]
    eo = eo_ref[...]
    oe = oe_ref[...]
    m = jnp.maximum
    top = m(m(ee[:, :oh, :ow], eo[:, :oh, :ow]), ee[:, :oh, 1:ow + 1])
    mid = m(m(oe[:, :oh, :ow], oo_ref[...]), oe[:, :oh, 1:ow + 1])
    bot = m(m(ee[:, 1:oh + 1, :ow], eo[:, 1:oh + 1, :ow]),
            ee[:, 1:oh + 1, 1:ow + 1])
    o_ref[...] = m(m(top, mid), bot)


def _conv_gemm(patches, w, b, s=None, t=None, *, tm=2048):
    G, M, K = patches.shape
    N = w.shape[2]
    Mp = _ru(M, tm)
    xp = jnp.pad(patches, ((0, 0), (0, Mp - M), (0, 0)))

    vec = pl.BlockSpec((G, 1, N), lambda i: (0, 0, 0))
    in_specs = [
        pl.BlockSpec((G, tm, K), lambda i: (0, i, 0)),
        pl.BlockSpec((G, K, N), lambda i: (0, 0, 0)),
        vec,
    ]
    args = [xp, w, b]
    body = functools.partial(_conv_body, groups=G)
    if s is not None:
        in_specs += [vec, vec]
        args += [s, t]
        body = functools.partial(_conv_bn_body, groups=G)

    out = pl.pallas_call(
        body,
        out_shape=jax.ShapeDtypeStruct((G, Mp, N), jnp.bfloat16),
        grid=(Mp // tm,),
        in_specs=in_specs,
        out_specs=pl.BlockSpec((G, tm, N), lambda i: (0, i, 0)),
        compiler_params=pltpu.CompilerParams(
            dimension_semantics=("parallel",),
            vmem_limit_bytes=_VMEM_LIMIT),
    )(*args)
    return out[:, :M, :]


def _fc(x, w, b, *, relu, out_dtype):
    M, K = x.shape
    Kp, Np = w.shape
    tn = 512 if Np % 512 == 0 else Np
    Mp = _ru(M, 16)
    xp = jnp.pad(x, ((0, Mp - M), (0, Kp - K)))

    out = pl.pallas_call(
        functools.partial(_fc_body, relu=relu),
        out_shape=jax.ShapeDtypeStruct((Mp, Np), out_dtype),
        grid=(Np // tn,),
        in_specs=[
            pl.BlockSpec((Mp, Kp), lambda j: (0, 0)),
            pl.BlockSpec((Kp, tn), lambda j: (0, j)),
            pl.BlockSpec((1, tn), lambda j: (0, j)),
        ],
        out_specs=pl.BlockSpec((Mp, tn), lambda j: (0, j)),
        compiler_params=pltpu.CompilerParams(
            dimension_semantics=("parallel",),
            vmem_limit_bytes=_VMEM_LIMIT),
    )(xp, w, b)
    return out[:M, :]


def _maxpool_3x3_s2(x, bt=16):
    B, H, W, C = x.shape
    if B % bt:
        bt = B
    OH = (H - 3) // 2 + 1
    OW = (W - 3) // 2 + 1
    ee = x[:, 0::2, 0::2]
    eo = x[:, 0::2, 1::2]
    oe = x[:, 1::2, 0::2]
    oo = x[:, 1::2, 1::2]

    def spec(a):
        return pl.BlockSpec((bt,) + a.shape[1:], lambda i: (i, 0, 0, 0))

    return pl.pallas_call(
        _pool_body,
        out_shape=jax.ShapeDtypeStruct((B, OH, OW, C), x.dtype),
        grid=(B // bt,),
        in_specs=[spec(ee), spec(eo), spec(oe), spec(oo)],
        out_specs=pl.BlockSpec((bt, OH, OW, C), lambda i: (i, 0, 0, 0)),
        compiler_params=pltpu.CompilerParams(
            dimension_semantics=("parallel",),
            vmem_limit_bytes=_VMEM_LIMIT),
    )(ee, eo, oe, oo)


def _im2col(x, kh, kw, stride, pad):
    if pad:
        x = jnp.pad(x, ((0, 0), (pad, pad), (pad, pad), (0, 0)))
    B, H, W, C = x.shape
    OH = (H - kh) // stride + 1
    OW = (W - kw) // stride + 1
    cols = []
    for dy in range(kh):
        for dx in range(kw):
            cols.append(x[:, dy:dy + stride * (OH - 1) + 1:stride,
                          dx:dx + stride * (OW - 1) + 1:stride, :])
    patches = jnp.stack(cols, axis=3)
    return patches.reshape(B * OH * OW, kh * kw * C), (OH, OW)


def _block_diag_w(w, groups, kg, ng, n_pad):
    g = w.shape[0]
    wt = w[:, :9 * kg, :ng].reshape(g, 9, kg, ng)
    wt = jnp.transpose(wt, (1, 0, 2, 3))
    eye = jnp.eye(g, dtype=w.dtype)
    blk = wt[:, :, :, None, :] * eye[None, :, None, :, None]
    blk = blk.reshape(9 * g * kg, g * ng)
    if n_pad:
        blk = jnp.pad(blk, ((0, 0), (0, n_pad)))
    return blk


def _cat_vec(v, ng, n_pad, fill=0.0):
    g = v.shape[0]
    flat = v[:, 0, :ng].reshape(1, g * ng)
    if n_pad:
        flat = jnp.pad(flat, ((0, 0), (0, n_pad)), constant_values=fill)
    return flat


def kernel(conv1_w, conv1_b, conv1_s, conv1_t,
           conv2_w, conv2_b, conv2_s, conv2_t,
           conv3_w, conv3_b, conv4_w, conv4_b, conv5_w, conv5_b,
           fc1_w, fc1_b, fc2_w, fc2_b, fc3_w, fc3_b, x):
    bf = jnp.bfloat16
    xh = jnp.transpose(x, (0, 2, 3, 1)).astype(bf)
    n = xh.shape[0]

    w1 = conv1_w[0, :363, :96].reshape(11, 11, 3, 96)
    w1 = jnp.pad(w1, ((0, 1), (0, 1), (0, 0), (0, 0)))
    w1 = w1.reshape(3, 4, 3, 4, 3, 96).transpose(0, 2, 1, 3, 4, 5)
    w1 = jnp.pad(w1.reshape(432, 96), ((0, 0), (0, 160))).astype(bf)
    b1 = jnp.pad(conv1_b[0, :, :96], ((0, 0), (0, 160)))
    s1 = jnp.pad(conv1_s[0, :, :96], ((0, 0), (0, 160)), constant_values=1.0)
    t1 = jnp.pad(conv1_t[0, :, :96], ((0, 0), (0, 160)))
    xb = jnp.pad(xh, ((0, 0), (0, 1), (0, 1), (0, 0)))
    xb = xb.reshape(n, 49, 4, 49, 4, 3).transpose(0, 1, 3, 2, 4, 5)
    xb = xb.reshape(n, 49, 49, 48)
    p1, (oh1, ow1) = _im2col(xb, 3, 3, 1, 0)
    c1 = _conv_gemm(p1[None], w1[None], b1[None], s1[None], t1[None])
    c1 = c1[0, :, :96].reshape(n, oh1, ow1, 96)
    c1 = _maxpool_3x3_s2(c1)

    w2 = _block_diag_w(conv2_w.astype(bf), 4, 24, 64, 0)
    b2 = _cat_vec(conv2_b, 64, 0)
    s2 = _cat_vec(conv2_s, 64, 0, fill=1.0)
    t2 = _cat_vec(conv2_t, 64, 0)
    p2, (oh, ow) = _im2col(c1, 3, 3, 1, 1)
    c2 = _conv_gemm(p2[None], w2[None], b2[None], s2[None], t2[None])
    c2 = c2[0].reshape(n, oh, ow, 256)
    c2 = _maxpool_3x3_s2(c2)

    w3 = _block_diag_w(conv3_w.astype(bf), 2, 128, 192, 0)
    b3 = _cat_vec(conv3_b, 192, 0)
    w4 = _block_diag_w(conv4_w.astype(bf), 4, 96, 96, 0)
    b4 = _cat_vec(conv4_b, 96, 0)
    w5 = _block_diag_w(conv5_w.astype(bf), 4, 96, 64, 0)
    b5 = _cat_vec(conv5_b, 64, 0)

    fr = jnp.arange(169) // 13
    fc = jnp.arange(169) % 13
    mask = ((fr >= 1) & (fr <= 11) & (fc >= 1) & (fc <= 11))
    mask = mask.astype(jnp.float32).reshape(169, 1)

    f = jnp.pad(c2, ((0, 0), (1, 1), (1, 1), (0, 0))).reshape(n, 169, 256)
    f = _flat_conv(f, w3, b3, mask)
    f = _flat_conv(f, w4, b4, mask)
    f = _flat_conv(f, w5, b5, mask)
    c5 = f.reshape(n, 13, 13, 256)[:, 1:12, 1:12]
    c5 = _maxpool_3x3_s2(c5)

    flat = jnp.transpose(c5, (0, 3, 1, 2)).reshape(n, -1)

    h1 = _fc(flat, fc1_w, fc1_b, relu=True, out_dtype=bf)
    h2 = _fc(h1, fc2_w, fc2_b, relu=True, out_dtype=bf)
    out = _fc(h2, fc3_w, fc3_b, relu=False, out_dtype=jnp.float32)
    return out[:, :_NUM_CLASSES]


# speedup vs baseline: 7.5134x; 1.3845x over previous
import functools

import jax
import jax.numpy as jnp
from jax.experimental import pallas as pl
from jax.experimental.pallas import tpu as pltpu

_NUM_CLASSES = 4
_VMEM_LIMIT = 48 * 1024 * 1024


def _ru(a, b):
    return ((a + b - 1) // b) * b


def _conv_body(x_ref, w_ref, b_ref, o_ref, *, groups):
    for g in range(groups):
        y = jnp.dot(x_ref[g], w_ref[g], preferred_element_type=jnp.float32)
        o_ref[g] = jnp.maximum(y + b_ref[g], 0.0).astype(o_ref.dtype)


def _conv_bn_body(x_ref, w_ref, b_ref, s_ref, t_ref, o_ref, *, groups):
    for g in range(groups):
        y = jnp.dot(x_ref[g], w_ref[g], preferred_element_type=jnp.float32)
        y = jnp.maximum(y + b_ref[g], 0.0)
        o_ref[g] = (y * s_ref[g] + t_ref[g]).astype(o_ref.dtype)


def _flat_conv_body(*refs, fw, bn=False):
    if bn:
        x_ref, w_ref, b_ref, s_ref, t_ref, m_ref, o_ref = refs
    else:
        x_ref, w_ref, b_ref, m_ref, o_ref = refs
    bt, ff, c = x_ref.shape
    r = bt * ff
    x2 = x_ref[...].reshape(r, c)
    xp = jnp.pad(x2, ((fw + 1, fw + 1), (0, 0)))
    acc = None
    for dy in (-1, 0, 1):
        s = fw + 1 + dy * fw - 1
        xc = jnp.concatenate(
            [xp[s:s + r], xp[s + 1:s + 1 + r], xp[s + 2:s + 2 + r]], axis=1)
        y = jnp.dot(xc, w_ref[(dy + 1) * 3 * c:(dy + 2) * 3 * c],
                    preferred_element_type=jnp.float32)
        acc = y if acc is None else acc + y
    y = jnp.maximum(acc + b_ref[...], 0.0)
    if bn:
        y = y * s_ref[...] + t_ref[...]
    y = y * jnp.tile(m_ref[...], (bt, 1))
    o_ref[...] = y.astype(o_ref.dtype).reshape(o_ref.shape)


def _flat_conv(x, w, b, mask, s=None, t=None, *, bt=16):
    n, ff, c = x.shape
    if n % bt:
        bt = n
    N = w.shape[1]
    fw = int(round(ff ** 0.5))
    vec = pl.BlockSpec((1, N), lambda i: (0, 0))
    in_specs = [
        pl.BlockSpec((bt, ff, c), lambda i: (i, 0, 0)),
        pl.BlockSpec(w.shape, lambda i: (0, 0)),
        vec,
    ]
    args = [x, w, b]
    bn = s is not None
    if bn:
        in_specs += [vec, vec]
        args += [s, t]
    in_specs.append(pl.BlockSpec((ff, 1), lambda i: (0, 0)))
    args.append(mask)
    body = functools.partial(_flat_conv_body, fw=fw, bn=bn)
    return pl.pallas_call(
        body,
        out_shape=jax.ShapeDtypeStruct((n, ff, N), jnp.bfloat16),
        grid=(n // bt,),
        in_specs=in_specs,
        out_specs=pl.BlockSpec((bt, ff, N), lambda i: (i, 0, 0)),
        compiler_params=pltpu.CompilerParams(
            dimension_semantics=("parallel",),
            vmem_limit_bytes=_VMEM_LIMIT),
    )(*args)


def _fc_body(x_ref, w_ref, b_ref, o_ref, *, relu):
    y = jnp.dot(x_ref[...], w_ref[...], preferred_element_type=jnp.float32)
    y = y + b_ref[...]
    if relu:
        y = jnp.maximum(y, 0.0)
    o_ref[...] = y.astype(o_ref.dtype)


def _pool_body(ee_ref, eo_ref, oe_ref, oo_ref, o_ref):
    oh = o_ref.shape[1]
    ow = o_ref.shape[2]
    ee = ee_ref[...]
    eo = eo_ref[...]
    oe = oe_ref[...]
    m = jnp.maximum
    top = m(m(ee[:, :oh, :ow], eo[:, :oh, :ow]), ee[:, :oh, 1:ow + 1])
    mid = m(m(oe[:, :oh, :ow], oo_ref[...]), oe[:, :oh, 1:ow + 1])
    bot = m(m(ee[:, 1:oh + 1, :ow], eo[:, 1:oh + 1, :ow]),
            ee[:, 1:oh + 1, 1:ow + 1])
    o_ref[...] = m(m(top, mid), bot)


def _conv_gemm(patches, w, b, s=None, t=None, *, tm=2048):
    G, M, K = patches.shape
    N = w.shape[2]
    Mp = _ru(M, tm)
    xp = jnp.pad(patches, ((0, 0), (0, Mp - M), (0, 0)))

    vec = pl.BlockSpec((G, 1, N), lambda i: (0, 0, 0))
    in_specs = [
        pl.BlockSpec((G, tm, K), lambda i: (0, i, 0)),
        pl.BlockSpec((G, K, N), lambda i: (0, 0, 0)),
        vec,
    ]
    args = [xp, w, b]
    body = functools.partial(_conv_body, groups=G)
    if s is not None:
        in_specs += [vec, vec]
        args += [s, t]
        body = functools.partial(_conv_bn_body, groups=G)

    out = pl.pallas_call(
        body,
        out_shape=jax.ShapeDtypeStruct((G, Mp, N), jnp.bfloat16),
        grid=(Mp // tm,),
        in_specs=in_specs,
        out_specs=pl.BlockSpec((G, tm, N), lambda i: (0, i, 0)),
        compiler_params=pltpu.CompilerParams(
            dimension_semantics=("parallel",),
            vmem_limit_bytes=_VMEM_LIMIT),
    )(*args)
    return out[:, :M, :]


def _fc(x, w, b, *, relu, out_dtype):
    M, K = x.shape
    Kp, Np = w.shape
    tn = 512 if Np % 512 == 0 else Np
    Mp = _ru(M, 16)
    xp = jnp.pad(x, ((0, Mp - M), (0, Kp - K)))

    out = pl.pallas_call(
        functools.partial(_fc_body, relu=relu),
        out_shape=jax.ShapeDtypeStruct((Mp, Np), out_dtype),
        grid=(Np // tn,),
        in_specs=[
            pl.BlockSpec((Mp, Kp), lambda j: (0, 0)),
            pl.BlockSpec((Kp, tn), lambda j: (0, j)),
            pl.BlockSpec((1, tn), lambda j: (0, j)),
        ],
        out_specs=pl.BlockSpec((Mp, tn), lambda j: (0, j)),
        compiler_params=pltpu.CompilerParams(
            dimension_semantics=("parallel",),
            vmem_limit_bytes=_VMEM_LIMIT),
    )(xp, w, b)
    return out[:M, :]


def _maxpool_3x3_s2(x, bt=16):
    B, H, W, C = x.shape
    if B % bt:
        bt = B
    OH = (H - 3) // 2 + 1
    OW = (W - 3) // 2 + 1
    ee = x[:, 0::2, 0::2]
    eo = x[:, 0::2, 1::2]
    oe = x[:, 1::2, 0::2]
    oo = x[:, 1::2, 1::2]

    def spec(a):
        return pl.BlockSpec((bt,) + a.shape[1:], lambda i: (i, 0, 0, 0))

    return pl.pallas_call(
        _pool_body,
        out_shape=jax.ShapeDtypeStruct((B, OH, OW, C), x.dtype),
        grid=(B // bt,),
        in_specs=[spec(ee), spec(eo), spec(oe), spec(oo)],
        out_specs=pl.BlockSpec((bt, OH, OW, C), lambda i: (i, 0, 0, 0)),
        compiler_params=pltpu.CompilerParams(
            dimension_semantics=("parallel",),
            vmem_limit_bytes=_VMEM_LIMIT),
    )(ee, eo, oe, oo)


def _im2col(x, kh, kw, stride, pad):
    if pad:
        x = jnp.pad(x, ((0, 0), (pad, pad), (pad, pad), (0, 0)))
    B, H, W, C = x.shape
    OH = (H - kh) // stride + 1
    OW = (W - kw) // stride + 1
    cols = []
    for dy in range(kh):
        for dx in range(kw):
            cols.append(x[:, dy:dy + stride * (OH - 1) + 1:stride,
                          dx:dx + stride * (OW - 1) + 1:stride, :])
    patches = jnp.stack(cols, axis=3)
    return patches.reshape(B * OH * OW, kh * kw * C), (OH, OW)


def _block_diag_w(w, groups, kg, ng, n_pad):
    g = w.shape[0]
    wt = w[:, :9 * kg, :ng].reshape(g, 9, kg, ng)
    wt = jnp.transpose(wt, (1, 0, 2, 3))
    eye = jnp.eye(g, dtype=w.dtype)
    blk = wt[:, :, :, None, :] * eye[None, :, None, :, None]
    blk = blk.reshape(9 * g * kg, g * ng)
    if n_pad:
        blk = jnp.pad(blk, ((0, 0), (0, n_pad)))
    return blk


def _cat_vec(v, ng, n_pad, fill=0.0):
    g = v.shape[0]
    flat = v[:, 0, :ng].reshape(1, g * ng)
    if n_pad:
        flat = jnp.pad(flat, ((0, 0), (0, n_pad)), constant_values=fill)
    return flat


def kernel(conv1_w, conv1_b, conv1_s, conv1_t,
           conv2_w, conv2_b, conv2_s, conv2_t,
           conv3_w, conv3_b, conv4_w, conv4_b, conv5_w, conv5_b,
           fc1_w, fc1_b, fc2_w, fc2_b, fc3_w, fc3_b, x):
    bf = jnp.bfloat16
    xh = jnp.transpose(x, (0, 2, 3, 1)).astype(bf)
    n = xh.shape[0]

    w1 = conv1_w[0, :363, :96].reshape(11, 11, 3, 96)
    w1 = jnp.pad(w1, ((0, 1), (0, 1), (0, 0), (0, 0)))
    w1 = w1.reshape(3, 4, 3, 4, 3, 96).transpose(0, 2, 1, 3, 4, 5)
    w1 = jnp.pad(w1.reshape(432, 96), ((0, 0), (0, 160))).astype(bf)
    b1 = jnp.pad(conv1_b[0, :, :96], ((0, 0), (0, 160)))
    s1 = jnp.pad(conv1_s[0, :, :96], ((0, 0), (0, 160)), constant_values=1.0)
    t1 = jnp.pad(conv1_t[0, :, :96], ((0, 0), (0, 160)))
    xb = jnp.pad(xh, ((0, 0), (0, 1), (0, 1), (0, 0)))
    xb = xb.reshape(n, 49, 4, 49, 4, 3).transpose(0, 1, 3, 2, 4, 5)
    xb = xb.reshape(n, 49, 49, 48)
    p1, (oh1, ow1) = _im2col(xb, 3, 3, 1, 0)
    c1 = _conv_gemm(p1[None], w1[None], b1[None], s1[None], t1[None])
    c1 = c1[0, :, :128].reshape(n, oh1, ow1, 128)
    c1 = _maxpool_3x3_s2(c1)

    w2 = _block_diag_w(conv2_w.astype(bf), 4, 24, 64, 0)
    w2 = jnp.pad(w2.reshape(9, 96, 256), ((0, 0), (0, 32), (0, 0)))
    w2 = w2.reshape(1152, 256)
    b2 = _cat_vec(conv2_b, 64, 0)
    s2 = _cat_vec(conv2_s, 64, 0, fill=1.0)
    t2 = _cat_vec(conv2_t, 64, 0)
    fr2 = jnp.arange(625) // 25
    fc2 = jnp.arange(625) % 25
    mask2 = ((fr2 >= 1) & (fr2 <= 23) & (fc2 >= 1) & (fc2 <= 23))
    mask2 = mask2.astype(jnp.float32).reshape(625, 1)
    f2 = jnp.pad(c1, ((0, 0), (1, 1), (1, 1), (0, 0))).reshape(n, 625, 128)
    f2 = _flat_conv(f2, w2, b2, mask2, s2, t2)
    c2 = f2.reshape(n, 25, 25, 256)[:, 1:24, 1:24]
    c2 = _maxpool_3x3_s2(c2)

    w3 = _block_diag_w(conv3_w.astype(bf), 2, 128, 192, 0)
    b3 = _cat_vec(conv3_b, 192, 0)
    w4 = _block_diag_w(conv4_w.astype(bf), 4, 96, 96, 0)
    b4 = _cat_vec(conv4_b, 96, 0)
    w5 = _block_diag_w(conv5_w.astype(bf), 4, 96, 64, 0)
    b5 = _cat_vec(conv5_b, 64, 0)

    fr = jnp.arange(169) // 13
    fc = jnp.arange(169) % 13
    mask = ((fr >= 1) & (fr <= 11) & (fc >= 1) & (fc <= 11))
    mask = mask.astype(jnp.float32).reshape(169, 1)

    f = jnp.pad(c2, ((0, 0), (1, 1), (1, 1), (0, 0))).reshape(n, 169, 256)
    f = _flat_conv(f, w3, b3, mask)
    f = _flat_conv(f, w4, b4, mask)
    f = _flat_conv(f, w5, b5, mask)
    c5 = f.reshape(n, 13, 13, 256)[:, 1:12, 1:12]
    c5 = _maxpool_3x3_s2(c5)

    flat = jnp.transpose(c5, (0, 3, 1, 2)).reshape(n, -1)

    h1 = _fc(flat, fc1_w, fc1_b, relu=True, out_dtype=bf)
    h2 = _fc(h1, fc2_w, fc2_b, relu=True, out_dtype=bf)
    out = _fc(h2, fc3_w, fc3_b, relu=False, out_dtype=jnp.float32)
    return out[:, :_NUM_CLASSES]


# speedup vs baseline: 36.9656x; 4.9199x over previous
import functools

import jax
import jax.numpy as jnp
from jax.experimental import pallas as pl
from jax.experimental.pallas import tpu as pltpu

_NUM_CLASSES = 4
_VMEM_LIMIT = 48 * 1024 * 1024


def _ru(a, b):
    return ((a + b - 1) // b) * b


def _conv_body(x_ref, w_ref, b_ref, o_ref, *, groups):
    for g in range(groups):
        y = jnp.dot(x_ref[g], w_ref[g], preferred_element_type=jnp.float32)
        o_ref[g] = jnp.maximum(y + b_ref[g], 0.0).astype(o_ref.dtype)


def _conv_bn_body(x_ref, w_ref, b_ref, s_ref, t_ref, o_ref, *, groups):
    for g in range(groups):
        y = jnp.dot(x_ref[g], w_ref[g], preferred_element_type=jnp.float32)
        y = jnp.maximum(y + b_ref[g], 0.0)
        o_ref[g] = (y * s_ref[g] + t_ref[g]).astype(o_ref.dtype)


def _flat_conv_body(*refs, fw, bn=False):
    if bn:
        x_ref, w_ref, b_ref, s_ref, t_ref, m_ref, o_ref = refs
    else:
        x_ref, w_ref, b_ref, m_ref, o_ref = refs
    bt, ff, c = x_ref.shape
    r = bt * ff
    x2 = x_ref[...].reshape(r, c)
    xp = jnp.pad(x2, ((fw + 1, fw + 1), (0, 0)))
    acc = None
    for dy in (-1, 0, 1):
        s = fw + 1 + dy * fw - 1
        xc = jnp.concatenate(
            [xp[s:s + r], xp[s + 1:s + 1 + r], xp[s + 2:s + 2 + r]], axis=1)
        y = jnp.dot(xc, w_ref[(dy + 1) * 3 * c:(dy + 2) * 3 * c],
                    preferred_element_type=jnp.float32)
        acc = y if acc is None else acc + y
    y = jnp.maximum(acc + b_ref[...], 0.0)
    if bn:
        y = y * s_ref[...] + t_ref[...]
    y = y * jnp.tile(m_ref[...], (bt, 1))
    o_ref[...] = y.astype(o_ref.dtype).reshape(o_ref.shape)


def _flat_conv(x, w, b, mask, s=None, t=None, *, bt=16):
    n, ff, c = x.shape
    if n % bt:
        bt = n
    N = w.shape[1]
    fw = int(round(ff ** 0.5))
    vec = pl.BlockSpec((1, N), lambda i: (0, 0))
    in_specs = [
        pl.BlockSpec((bt, ff, c), lambda i: (i, 0, 0)),
        pl.BlockSpec(w.shape, lambda i: (0, 0)),
        vec,
    ]
    args = [x, w, b]
    bn = s is not None
    if bn:
        in_specs += [vec, vec]
        args += [s, t]
    in_specs.append(pl.BlockSpec((ff, 1), lambda i: (0, 0)))
    args.append(mask)
    body = functools.partial(_flat_conv_body, fw=fw, bn=bn)
    return pl.pallas_call(
        body,
        out_shape=jax.ShapeDtypeStruct((n, ff, N), jnp.bfloat16),
        grid=(n // bt,),
        in_specs=in_specs,
        out_specs=pl.BlockSpec((bt, ff, N), lambda i: (i, 0, 0)),
        compiler_params=pltpu.CompilerParams(
            dimension_semantics=("parallel",),
            vmem_limit_bytes=_VMEM_LIMIT),
    )(*args)


def _fc_body(x_ref, w_ref, b_ref, o_ref, *, relu):
    y = jnp.dot(x_ref[...], w_ref[...], preferred_element_type=jnp.float32)
    y = y + b_ref[...]
    if relu:
        y = jnp.maximum(y, 0.0)
    o_ref[...] = y.astype(o_ref.dtype)


def _pool_body(ee_ref, eo_ref, oe_ref, oo_ref, o_ref):
    oh = o_ref.shape[1]
    ow = o_ref.shape[2]
    ee = ee_ref[...]
    eo = eo_ref[...]
    oe = oe_ref[...]
    m = jnp.maximum
    top = m(m(ee[:, :oh, :ow], eo[:, :oh, :ow]), ee[:, :oh, 1:ow + 1])
    mid = m(m(oe[:, :oh, :ow], oo_ref[...]), oe[:, :oh, 1:ow + 1])
    bot = m(m(ee[:, 1:oh + 1, :ow], eo[:, 1:oh + 1, :ow]),
            ee[:, 1:oh + 1, 1:ow + 1])
    o_ref[...] = m(m(top, mid), bot)


def _conv_gemm(patches, w, b, s=None, t=None, *, tm=2048):
    G, M, K = patches.shape
    N = w.shape[2]
    Mp = _ru(M, tm)
    xp = jnp.pad(patches, ((0, 0), (0, Mp - M), (0, 0)))

    vec = pl.BlockSpec((G, 1, N), lambda i: (0, 0, 0))
    in_specs = [
        pl.BlockSpec((G, tm, K), lambda i: (0, i, 0)),
        pl.BlockSpec((G, K, N), lambda i: (0, 0, 0)),
        vec,
    ]
    args = [xp, w, b]
    body = functools.partial(_conv_body, groups=G)
    if s is not None:
        in_specs += [vec, vec]
        args += [s, t]
        body = functools.partial(_conv_bn_body, groups=G)

    out = pl.pallas_call(
        body,
        out_shape=jax.ShapeDtypeStruct((G, Mp, N), jnp.bfloat16),
        grid=(Mp // tm,),
        in_specs=in_specs,
        out_specs=pl.BlockSpec((G, tm, N), lambda i: (0, i, 0)),
        compiler_params=pltpu.CompilerParams(
            dimension_semantics=("parallel",),
            vmem_limit_bytes=_VMEM_LIMIT),
    )(*args)
    return out[:, :M, :]


def _fc(x, w, b, *, relu, out_dtype):
    M, K = x.shape
    Kp, Np = w.shape
    tn = 512 if Np % 512 == 0 else Np
    Mp = _ru(M, 16)
    xp = jnp.pad(x, ((0, Mp - M), (0, Kp - K)))

    out = pl.pallas_call(
        functools.partial(_fc_body, relu=relu),
        out_shape=jax.ShapeDtypeStruct((Mp, Np), out_dtype),
        grid=(Np // tn,),
        in_specs=[
            pl.BlockSpec((Mp, Kp), lambda j: (0, 0)),
            pl.BlockSpec((Kp, tn), lambda j: (0, j)),
            pl.BlockSpec((1, tn), lambda j: (0, j)),
        ],
        out_specs=pl.BlockSpec((Mp, tn), lambda j: (0, j)),
        compiler_params=pltpu.CompilerParams(
            dimension_semantics=("parallel",),
            vmem_limit_bytes=_VMEM_LIMIT),
    )(xp, w, b)
    return out[:M, :]


def _maxpool_3x3_s2(x, bt=16):
    B, H, W, C = x.shape
    if B % bt:
        bt = B
    OH = (H - 3) // 2 + 1
    OW = (W - 3) // 2 + 1
    ee = x[:, 0::2, 0::2]
    eo = x[:, 0::2, 1::2]
    oe = x[:, 1::2, 0::2]
    oo = x[:, 1::2, 1::2]

    def spec(a):
        return pl.BlockSpec((bt,) + a.shape[1:], lambda i: (i, 0, 0, 0))

    return pl.pallas_call(
        _pool_body,
        out_shape=jax.ShapeDtypeStruct((B, OH, OW, C), x.dtype),
        grid=(B // bt,),
        in_specs=[spec(ee), spec(eo), spec(oe), spec(oo)],
        out_specs=pl.BlockSpec((bt, OH, OW, C), lambda i: (i, 0, 0, 0)),
        compiler_params=pltpu.CompilerParams(
            dimension_semantics=("parallel",),
            vmem_limit_bytes=_VMEM_LIMIT),
    )(ee, eo, oe, oo)


def _im2col(x, kh, kw, stride, pad):
    if pad:
        x = jnp.pad(x, ((0, 0), (pad, pad), (pad, pad), (0, 0)))
    B, H, W, C = x.shape
    OH = (H - kh) // stride + 1
    OW = (W - kw) // stride + 1
    cols = []
    for dy in range(kh):
        for dx in range(kw):
            cols.append(x[:, dy:dy + stride * (OH - 1) + 1:stride,
                          dx:dx + stride * (OW - 1) + 1:stride, :])
    patches = jnp.stack(cols, axis=3)
    return patches.reshape(B * OH * OW, kh * kw * C), (OH, OW)


def _block_diag_w(w, groups, kg, ng, n_pad):
    g = w.shape[0]
    wt = w[:, :9 * kg, :ng].reshape(g, 9, kg, ng)
    wt = jnp.transpose(wt, (1, 0, 2, 3))
    eye = jnp.eye(g, dtype=w.dtype)
    blk = wt[:, :, :, None, :] * eye[None, :, None, :, None]
    blk = blk.reshape(9 * g * kg, g * ng)
    if n_pad:
        blk = jnp.pad(blk, ((0, 0), (0, n_pad)))
    return blk


def _cat_vec(v, ng, n_pad, fill=0.0):
    g = v.shape[0]
    flat = v[:, 0, :ng].reshape(1, g * ng)
    if n_pad:
        flat = jnp.pad(flat, ((0, 0), (0, n_pad)), constant_values=fill)
    return flat


def kernel(conv1_w, conv1_b, conv1_s, conv1_t,
           conv2_w, conv2_b, conv2_s, conv2_t,
           conv3_w, conv3_b, conv4_w, conv4_b, conv5_w, conv5_b,
           fc1_w, fc1_b, fc2_w, fc2_b, fc3_w, fc3_b, x):
    bf = jnp.bfloat16
    n = x.shape[0]

    w1 = conv1_w[0, :363, :96].reshape(11, 11, 3, 96)
    w1 = jnp.pad(w1, ((0, 1), (0, 1), (0, 0), (0, 0)))
    w1 = w1.reshape(3, 4, 3, 4, 3, 96).transpose(0, 2, 1, 3, 4, 5)
    w1 = jnp.pad(w1.reshape(9, 48, 96), ((0, 0), (0, 80), (0, 160)))
    w1 = w1.reshape(1152, 256).astype(bf)
    b1 = jnp.pad(conv1_b[0, :, :96], ((0, 0), (0, 160)))
    s1 = jnp.pad(conv1_s[0, :, :96], ((0, 0), (0, 160)), constant_values=1.0)
    t1 = jnp.pad(conv1_t[0, :, :96], ((0, 0), (0, 160)))
    xb = jnp.pad(x, ((0, 0), (0, 0), (0, 1), (0, 1))).astype(bf)
    xb = xb.reshape(n, 3, 49, 4, 49, 4).transpose(0, 2, 4, 3, 5, 1)
    xb = jnp.pad(xb.reshape(n, 49, 49, 48), ((0, 0),) * 3 + ((0, 80),))
    ones1 = jnp.ones((2401, 1), jnp.float32)
    f1 = _flat_conv(xb.reshape(n, 2401, 128), w1, b1, ones1, s1, t1, bt=4)
    c1 = f1.reshape(n, 49, 49, 256)[:, 1:48, 1:48, :128]
    c1 = _maxpool_3x3_s2(c1)

    w2 = _block_diag_w(conv2_w.astype(bf), 4, 24, 64, 0)
    w2 = jnp.pad(w2.reshape(9, 96, 256), ((0, 0), (0, 32), (0, 0)))
    w2 = w2.reshape(1152, 256)
    b2 = _cat_vec(conv2_b, 64, 0)
    s2 = _cat_vec(conv2_s, 64, 0, fill=1.0)
    t2 = _cat_vec(conv2_t, 64, 0)
    fr2 = jnp.arange(625) // 25
    fc2 = jnp.arange(625) % 25
    mask2 = ((fr2 >= 1) & (fr2 <= 23) & (fc2 >= 1) & (fc2 <= 23))
    mask2 = mask2.astype(jnp.float32).reshape(625, 1)
    f2 = jnp.pad(c1, ((0, 0), (1, 1), (1, 1), (0, 0))).reshape(n, 625, 128)
    f2 = _flat_conv(f2, w2, b2, mask2, s2, t2)
    c2 = f2.reshape(n, 25, 25, 256)[:, 1:24, 1:24]
    c2 = _maxpool_3x3_s2(c2)

    w3 = _block_diag_w(conv3_w.astype(bf), 2, 128, 192, 0)
    b3 = _cat_vec(conv3_b, 192, 0)
    w4 = _block_diag_w(conv4_w.astype(bf), 4, 96, 96, 0)
    b4 = _cat_vec(conv4_b, 96, 0)
    w5 = _block_diag_w(conv5_w.astype(bf), 4, 96, 64, 0)
    b5 = _cat_vec(conv5_b, 64, 0)

    fr = jnp.arange(169) // 13
    fc = jnp.arange(169) % 13
    mask = ((fr >= 1) & (fr <= 11) & (fc >= 1) & (fc <= 11))
    mask = mask.astype(jnp.float32).reshape(169, 1)

    f = jnp.pad(c2, ((0, 0), (1, 1), (1, 1), (0, 0))).reshape(n, 169, 256)
    f = _flat_conv(f, w3, b3, mask)
    f = _flat_conv(f, w4, b4, mask)
    f = _flat_conv(f, w5, b5, mask)
    c5 = f.reshape(n, 13, 13, 256)[:, 1:12, 1:12]
    c5 = _maxpool_3x3_s2(c5)

    flat = jnp.transpose(c5, (0, 3, 1, 2)).reshape(n, -1)

    h1 = _fc(flat, fc1_w, fc1_b, relu=True, out_dtype=bf)
    h2 = _fc(h1, fc2_w, fc2_b, relu=True, out_dtype=bf)
    out = _fc(h2, fc3_w, fc3_b, relu=False, out_dtype=jnp.float32)
    return out[:, :_NUM_CLASSES]
